# Initial kernel scaffold; baseline (speedup 1.0000x reference)
import jax
import jax.numpy as jnp
from jax.experimental import pallas as pl


def kernel(user_emb, item_emb, user_ids, item_ids):
    raise NotImplementedError("write your pallas kernel here")


# speedup vs baseline: 3.0902x; 3.0902x over previous
import functools

import jax
import jax.numpy as jnp
from jax import lax
from jax.experimental import pallas as pl
from jax.experimental.pallas import tpu as pltpu
from jax.experimental.pallas import tpu_sc as plsc

NUM_USERS = 100000
NUM_ITEMS = 100000
EMBED_DIM = 64
BATCH = 4096

_INFO = plsc.get_sparse_core_info()
_NC, _NS, _L = _INFO.num_cores, _INFO.num_subcores, _INFO.num_lanes
_NW = _NC * _NS
_BPW = BATCH // _NW
_GROUPS = _BPW // _L


def _sc_kernel(user_hbm, item_hbm, uid_hbm, iid_hbm, out_hbm,
               uid_v, iid_v, uidx2_v, iidx2_v,
               cu1_v, cu2_v, ci1_v, ci2_v,
               rows_ua, rows_ub, rows_ia, rows_ib, out_v, sem):
    wid = lax.axis_index("s") * _NC + lax.axis_index("c")
    base = wid * _BPW

    pltpu.sync_copy(uid_hbm.at[pl.ds(base, _BPW)], uid_v)
    pltpu.sync_copy(iid_hbm.at[pl.ds(base, _BPW)], iid_v)

    for g in range(_GROUPS):
        sl = pl.ds(g * _L, _L)
        u = uid_v[sl]
        su = jnp.logical_and(jnp.equal(jnp.remainder(u, 1500), 0),
                             u <= 94500)
        uidx2_v[sl] = jnp.minimum(u + 3, NUM_ITEMS - 1)
        half = jnp.full((_L,), 0.5, jnp.float32)
        quarter = jnp.full((_L,), 0.25, jnp.float32)
        zero = jnp.zeros((_L,), jnp.float32)
        cu1_v[sl] = jnp.where(su, half, quarter)
        cu2_v[sl] = jnp.where(su, half, zero)

        i = iid_v[sl]
        si = jnp.logical_and(
            jnp.logical_and(jnp.equal(jnp.remainder(i - 3, 1500), 0), i >= 3),
            i <= 94503)
        iidx2_v[sl] = jnp.maximum(i - 3, 0)
        ci1_v[sl] = jnp.where(si, half, zero)
        ci2_v[sl] = jnp.where(si, half, quarter)

    c1 = pltpu.async_copy(user_hbm.at[uid_v], rows_ua, sem)
    c2 = pltpu.async_copy(item_hbm.at[uidx2_v], rows_ub, sem)
    c3 = pltpu.async_copy(user_hbm.at[iidx2_v], rows_ia, sem)
    c4 = pltpu.async_copy(item_hbm.at[iid_v], rows_ib, sem)
    c1.wait(); c2.wait(); c3.wait(); c4.wait()

    for g in range(_GROUPS):
        sl = pl.ds(g * _L, _L)
        row_idx = jax.lax.iota(jnp.int32, _L) + g * _L
        cu1 = cu1_v[sl]
        cu2 = cu2_v[sl]
        ci1 = ci1_v[sl]
        ci2 = ci2_v[sl]

        def body(d, acc):
            col = jnp.full((_L,), d, jnp.int32)
            ua = plsc.load_gather(rows_ua, [row_idx, col])
            ub = plsc.load_gather(rows_ub, [row_idx, col])
            ia = plsc.load_gather(rows_ia, [row_idx, col])
            ib = plsc.load_gather(rows_ib, [row_idx, col])
            ue = cu1 * ua + cu2 * ub
            ie = ci1 * ia + ci2 * ib
            return acc + ue * ie

        out_v[sl] = lax.fori_loop(0, EMBED_DIM, body,
                                  jnp.zeros((_L,), jnp.float32))

    pltpu.sync_copy(out_v, out_hbm.at[pl.ds(base, _BPW)])


@jax.jit
def _run(user_emb, item_emb, user_ids, item_ids):
    mesh = plsc.VectorSubcoreMesh(core_axis_name="c", subcore_axis_name="s")
    kern = functools.partial(
        pl.kernel,
        mesh=mesh,
        compiler_params=pltpu.CompilerParams(
            needs_layout_passes=False, use_tc_tiling_on_sc=False),
        out_type=jax.ShapeDtypeStruct((BATCH,), jnp.float32),
        scratch_types=[
            pltpu.VMEM((_BPW,), jnp.int32),
            pltpu.VMEM((_BPW,), jnp.int32),
            pltpu.VMEM((_BPW,), jnp.int32),
            pltpu.VMEM((_BPW,), jnp.int32),
            pltpu.VMEM((_BPW,), jnp.float32),
            pltpu.VMEM((_BPW,), jnp.float32),
            pltpu.VMEM((_BPW,), jnp.float32),
            pltpu.VMEM((_BPW,), jnp.float32),
            pltpu.VMEM((_BPW, EMBED_DIM), jnp.float32),
            pltpu.VMEM((_BPW, EMBED_DIM), jnp.float32),
            pltpu.VMEM((_BPW, EMBED_DIM), jnp.float32),
            pltpu.VMEM((_BPW, EMBED_DIM), jnp.float32),
            pltpu.VMEM((_BPW,), jnp.float32),
            pltpu.SemaphoreType.DMA,
        ],
    )(_sc_kernel)
    return kern(user_emb, item_emb, user_ids, item_ids)


def kernel(user_emb, item_emb, user_ids, item_ids):
    return _run(user_emb, item_emb,
                user_ids.astype(jnp.int32), item_ids.astype(jnp.int32))


# speedup vs baseline: 4.5313x; 1.4663x over previous
import functools

import jax
import jax.numpy as jnp
from jax import lax
from jax.experimental import pallas as pl
from jax.experimental.pallas import tpu as pltpu
from jax.experimental.pallas import tpu_sc as plsc

NUM_USERS = 100000
NUM_ITEMS = 100000
EMBED_DIM = 64
BATCH = 4096

_INFO = plsc.get_sparse_core_info()
_NC, _NS, _L = _INFO.num_cores, _INFO.num_subcores, _INFO.num_lanes
_NW = _NC * _NS
_BPW = BATCH // _NW
_GROUPS = _BPW // _L
_WAVE = 16
_NWAVES = _BPW // _WAVE


def _sc_kernel(user_hbm, item_hbm, uid_hbm, iid_hbm, out_hbm,
               uid_v, iid_v, ga_v, gb_v, gc_v, gd_v,
               sa_v, sb_v, sc_v, sd_v,
               cu1_v, cu2_v, ci1_v, ci2_v,
               rows_ua, rows_ub, rows_ia, rows_ib, out_v, sem):
    wid = lax.axis_index("s") * _NC + lax.axis_index("c")
    base = wid * _BPW

    pltpu.sync_copy(uid_hbm.at[pl.ds(base, _BPW)], uid_v)
    pltpu.sync_copy(iid_hbm.at[pl.ds(base, _BPW)], iid_v)

    for g in range(_GROUPS):
        sl = pl.ds(g * _L, _L)
        u = uid_v[sl]
        su = jnp.logical_and(jnp.equal(jnp.remainder(u, 1500), 0),
                             u <= 94500)
        ub_idx = jnp.minimum(u + 3, NUM_ITEMS - 1)
        ga_v[sl] = jnp.right_shift(u, 3)
        sa_v[sl] = jnp.bitwise_and(u, 7)
        gb_v[sl] = jnp.right_shift(ub_idx, 3)
        sb_v[sl] = jnp.bitwise_and(ub_idx, 7)
        half = jnp.full((_L,), 0.5, jnp.float32)
        quarter = jnp.full((_L,), 0.25, jnp.float32)
        zero = jnp.zeros((_L,), jnp.float32)
        cu1_v[sl] = jnp.where(su, half, quarter)
        cu2_v[sl] = jnp.where(su, half, zero)

        i = iid_v[sl]
        si = jnp.logical_and(
            jnp.logical_and(jnp.equal(jnp.remainder(i - 3, 1500), 0), i >= 3),
            i <= 94503)
        ia_idx = jnp.maximum(i - 3, 0)
        gc_v[sl] = jnp.right_shift(ia_idx, 3)
        sc_v[sl] = jnp.bitwise_and(ia_idx, 7)
        gd_v[sl] = jnp.right_shift(i, 3)
        sd_v[sl] = jnp.bitwise_and(i, 7)
        ci1_v[sl] = jnp.where(si, half, zero)
        ci2_v[sl] = jnp.where(si, half, quarter)

    lane = lax.iota(jnp.int32, _L)

    for w in range(_NWAVES):
        wbase = w * _WAVE

        def fire(j, _):
            b = wbase + j
            ga = ga_v[pl.ds(b, _L)][0]
            gb = gb_v[pl.ds(b, _L)][0]
            gc = gc_v[pl.ds(b, _L)][0]
            gd = gd_v[pl.ds(b, _L)][0]
            pltpu.async_copy(user_hbm.at[pl.ds(ga, 1)],
                             rows_ua.at[pl.ds(j, 1)], sem)
            pltpu.async_copy(item_hbm.at[pl.ds(gb, 1)],
                             rows_ub.at[pl.ds(j, 1)], sem)
            pltpu.async_copy(user_hbm.at[pl.ds(gc, 1)],
                             rows_ia.at[pl.ds(j, 1)], sem)
            pltpu.async_copy(item_hbm.at[pl.ds(gd, 1)],
                             rows_ib.at[pl.ds(j, 1)], sem)
            return ()

        lax.fori_loop(0, _WAVE, fire, ())
        for buf in (rows_ua, rows_ub, rows_ia, rows_ib):
            pltpu.make_async_copy(user_hbm.at[pl.ds(0, _WAVE)], buf, sem).wait()

        for g in range(_WAVE // _L):
            sl = pl.ds(wbase + g * _L, _L)
            lrow = lane + g * _L
            cu1 = cu1_v[sl]
            cu2 = cu2_v[sl]
            ci1 = ci1_v[sl]
            ci2 = ci2_v[sl]
            sa = sa_v[sl]
            sb = sb_v[sl]
            sc = sc_v[sl]
            sd = sd_v[sl]

            def body(d, acc):
                col = jnp.bitwise_and(lane + d, EMBED_DIM - 1)
                ua = plsc.load_gather(rows_ua, [lrow, sa, col])
                ub = plsc.load_gather(rows_ub, [lrow, sb, col])
                ia = plsc.load_gather(rows_ia, [lrow, sc, col])
                ib = plsc.load_gather(rows_ib, [lrow, sd, col])
                ue = cu1 * ua + cu2 * ub
                ie = ci1 * ia + ci2 * ib
                return acc + ue * ie

            out_v[sl] = lax.fori_loop(0, EMBED_DIM, body,
                                      jnp.zeros((_L,), jnp.float32))

    pltpu.sync_copy(out_v, out_hbm.at[pl.ds(base, _BPW)])


@jax.jit
def _run(user_emb, item_emb, user_ids, item_ids):
    mesh = plsc.VectorSubcoreMesh(core_axis_name="c", subcore_axis_name="s")
    kern = functools.partial(
        pl.kernel,
        mesh=mesh,
        compiler_params=pltpu.CompilerParams(
            needs_layout_passes=False, use_tc_tiling_on_sc=True),
        out_type=jax.ShapeDtypeStruct((BATCH,), jnp.float32),
        scratch_types=[
            pltpu.VMEM((_BPW,), jnp.int32),
            pltpu.VMEM((_BPW,), jnp.int32),
            pltpu.VMEM((_BPW + _L,), jnp.int32),
            pltpu.VMEM((_BPW + _L,), jnp.int32),
            pltpu.VMEM((_BPW + _L,), jnp.int32),
            pltpu.VMEM((_BPW + _L,), jnp.int32),
            pltpu.VMEM((_BPW,), jnp.int32),
            pltpu.VMEM((_BPW,), jnp.int32),
            pltpu.VMEM((_BPW,), jnp.int32),
            pltpu.VMEM((_BPW,), jnp.int32),
            pltpu.VMEM((_BPW,), jnp.float32),
            pltpu.VMEM((_BPW,), jnp.float32),
            pltpu.VMEM((_BPW,), jnp.float32),
            pltpu.VMEM((_BPW,), jnp.float32),
            pltpu.VMEM((_WAVE, 8, EMBED_DIM), jnp.float32),
            pltpu.VMEM((_WAVE, 8, EMBED_DIM), jnp.float32),
            pltpu.VMEM((_WAVE, 8, EMBED_DIM), jnp.float32),
            pltpu.VMEM((_WAVE, 8, EMBED_DIM), jnp.float32),
            pltpu.VMEM((_BPW,), jnp.float32),
            pltpu.SemaphoreType.DMA,
        ],
    )(_sc_kernel)
    u3 = user_emb.reshape(NUM_USERS // 8, 8, EMBED_DIM)
    i3 = item_emb.reshape(NUM_ITEMS // 8, 8, EMBED_DIM)
    return kern(u3, i3, user_ids, item_ids)


def kernel(user_emb, item_emb, user_ids, item_ids):
    return _run(user_emb, item_emb,
                user_ids.astype(jnp.int32), item_ids.astype(jnp.int32))


# speedup vs baseline: 4.8903x; 1.0792x over previous
import functools

import jax
import jax.numpy as jnp
from jax import lax
from jax.experimental import pallas as pl
from jax.experimental.pallas import tpu as pltpu
from jax.experimental.pallas import tpu_sc as plsc

NUM_USERS = 100000
NUM_ITEMS = 100000
EMBED_DIM = 64
BATCH = 4096

_INFO = plsc.get_sparse_core_info()
_NC, _NS, _L = _INFO.num_cores, _INFO.num_subcores, _INFO.num_lanes
_NW = _NC * _NS
_BPW = BATCH // _NW
_GROUPS = _BPW // _L


def _sc_kernel(user_hbm, item_hbm, uid_hbm, iid_hbm, out_hbm,
               uid_v, iid_v, ga_v, gb_v, gc_v, gd_v,
               cu1_v, cu2_v, ci1_v, ci2_v,
               rows_ua, rows_ub, rows_ia, rows_ib, out_v, sem):
    wid = lax.axis_index("s") * _NC + lax.axis_index("c")
    base = wid * _BPW

    pltpu.sync_copy(uid_hbm.at[pl.ds(base, _BPW)], uid_v)
    pltpu.sync_copy(iid_hbm.at[pl.ds(base, _BPW)], iid_v)

    for g in range(_GROUPS):
        sl = pl.ds(g * _L, _L)
        u = uid_v[sl]
        su = jnp.logical_and(jnp.equal(jnp.remainder(u, 1500), 0),
                             u <= 94500)
        ga_v[sl] = u
        gb_v[sl] = jnp.minimum(u + 3, NUM_ITEMS - 1)
        half = jnp.full((_L,), 0.5, jnp.float32)
        quarter = jnp.full((_L,), 0.25, jnp.float32)
        zero = jnp.zeros((_L,), jnp.float32)
        cu1_v[sl] = jnp.where(su, half, quarter)
        cu2_v[sl] = jnp.where(su, half, zero)

        i = iid_v[sl]
        si = jnp.logical_and(
            jnp.logical_and(jnp.equal(jnp.remainder(i - 3, 1500), 0), i >= 3),
            i <= 94503)
        gc_v[sl] = jnp.maximum(i - 3, 0)
        gd_v[sl] = i
        ci1_v[sl] = jnp.where(si, half, zero)
        ci2_v[sl] = jnp.where(si, half, quarter)

    def fire(b, _):
        ga = ga_v[pl.ds(b, _L)][0]
        gb = gb_v[pl.ds(b, _L)][0]
        gc = gc_v[pl.ds(b, _L)][0]
        gd = gd_v[pl.ds(b, _L)][0]
        pltpu.async_copy(user_hbm.at[pl.ds(ga, 1)],
                         rows_ua.at[pl.ds(b, 1)], sem)
        pltpu.async_copy(item_hbm.at[pl.ds(gb, 1)],
                         rows_ub.at[pl.ds(b, 1)], sem)
        pltpu.async_copy(user_hbm.at[pl.ds(gc, 1)],
                         rows_ia.at[pl.ds(b, 1)], sem)
        pltpu.async_copy(item_hbm.at[pl.ds(gd, 1)],
                         rows_ib.at[pl.ds(b, 1)], sem)
        return ()

    lax.fori_loop(0, _BPW, fire, ())
    for buf in (rows_ua, rows_ub, rows_ia, rows_ib):
        pltpu.make_async_copy(user_hbm.at[pl.ds(0, _BPW)], buf, sem).wait()

    lane = lax.iota(jnp.int32, _L)
    for g in range(_GROUPS):
        sl = pl.ds(g * _L, _L)
        lrow = lane + g * _L
        cu1 = cu1_v[sl]
        cu2 = cu2_v[sl]
        ci1 = ci1_v[sl]
        ci2 = ci2_v[sl]

        def body(d, acc):
            col = jnp.bitwise_and(lane + d, EMBED_DIM - 1)
            ua = plsc.load_gather(rows_ua, [lrow, col])
            ub = plsc.load_gather(rows_ub, [lrow, col])
            ia = plsc.load_gather(rows_ia, [lrow, col])
            ib = plsc.load_gather(rows_ib, [lrow, col])
            ue = cu1 * ua + cu2 * ub
            ie = ci1 * ia + ci2 * ib
            return acc + ue * ie

        out_v[sl] = lax.fori_loop(0, EMBED_DIM, body,
                                  jnp.zeros((_L,), jnp.float32))

    pltpu.sync_copy(out_v, out_hbm.at[pl.ds(base, _BPW)])


@jax.jit
def _run(user_emb, item_emb, user_ids, item_ids):
    mesh = plsc.VectorSubcoreMesh(core_axis_name="c", subcore_axis_name="s")
    kern = functools.partial(
        pl.kernel,
        mesh=mesh,
        compiler_params=pltpu.CompilerParams(
            needs_layout_passes=False, use_tc_tiling_on_sc=True),
        out_type=jax.ShapeDtypeStruct((BATCH,), jnp.float32),
        scratch_types=[
            pltpu.VMEM((_BPW,), jnp.int32),
            pltpu.VMEM((_BPW,), jnp.int32),
            pltpu.VMEM((_BPW + _L,), jnp.int32),
            pltpu.VMEM((_BPW + _L,), jnp.int32),
            pltpu.VMEM((_BPW + _L,), jnp.int32),
            pltpu.VMEM((_BPW + _L,), jnp.int32),
            pltpu.VMEM((_BPW,), jnp.float32),
            pltpu.VMEM((_BPW,), jnp.float32),
            pltpu.VMEM((_BPW,), jnp.float32),
            pltpu.VMEM((_BPW,), jnp.float32),
            pltpu.VMEM((_BPW, EMBED_DIM), jnp.float32),
            pltpu.VMEM((_BPW, EMBED_DIM), jnp.float32),
            pltpu.VMEM((_BPW, EMBED_DIM), jnp.float32),
            pltpu.VMEM((_BPW, EMBED_DIM), jnp.float32),
            pltpu.VMEM((_BPW,), jnp.float32),
            pltpu.SemaphoreType.DMA,
        ],
    )(_sc_kernel)
    return kern(user_emb, item_emb, user_ids, item_ids)


def kernel(user_emb, item_emb, user_ids, item_ids):
    return _run(user_emb, item_emb,
                user_ids.astype(jnp.int32), item_ids.astype(jnp.int32))
